# baseline (speedup 1.0000x reference)
import functools

import jax
import jax.numpy as jnp
from jax import lax
from jax.experimental import pallas as pl
from jax.experimental.pallas import tpu as pltpu
from jax.experimental.pallas import tpu_sc as plsc

N = 4 * 8192
D = 512
NC, NS = 2, 16
NW = NC * NS
ROWS_PER_W = N // NW
CHUNK = 64
NCHUNK = ROWS_PER_W // CHUNK
GROUPS = CHUNK // 16
UNROLL = 8

_mesh = plsc.VectorSubcoreMesh(core_axis_name="c", subcore_axis_name="s")


@functools.partial(
    pl.kernel,
    mesh=_mesh,
    out_type=jax.ShapeDtypeStruct((N * D,), jnp.float32),
    scratch_types=[
        pltpu.VMEM((ROWS_PER_W,), jnp.int32),
        pltpu.VMEM((3 * D,), jnp.float32),
        pltpu.VMEM((CHUNK * D,), jnp.float32),
        pltpu.VMEM((CHUNK * D,), jnp.float32),
        pltpu.SemaphoreType.DMA,
    ],
    compiler_params=pltpu.CompilerParams(needs_layout_passes=False),
)
def _sc_embed(idx_hbm, table_hbm, out_hbm, idx_v, tab_v, rows_a, rows_b, ssem):
    wid = lax.axis_index("s") * NC + lax.axis_index("c")
    base = wid * ROWS_PER_W
    pltpu.sync_copy(idx_hbm.at[pl.ds(base, ROWS_PER_W)], idx_v)
    pltpu.sync_copy(table_hbm, tab_v)

    lane = jnp.arange(16, dtype=jnp.int32)
    lane_d = lane * D

    bufs = (rows_a, rows_b)
    pending = [None, None]
    for ci in range(NCHUNK):
        buf = bufs[ci % 2]
        if pending[ci % 2] is not None:
            pending[ci % 2].wait()
        for g in range(GROUPS):
            idxvec = idx_v[pl.ds(ci * CHUNK + g * 16, 16)]
            tbase = idxvec * D
            obase = lane_d + g * (16 * D)

            def col_block(c0, _, tbase=tbase, obase=obase, buf=buf):
                c = c0 * UNROLL
                for k in range(UNROLL):
                    vals = plsc.load_gather(tab_v, [tbase + (c + k)])
                    plsc.store_scatter(buf, [obase + (c + k)], vals)
                return _

            lax.fori_loop(0, D // UNROLL, col_block, 0)
        pending[ci % 2] = pltpu.async_copy(
            buf, out_hbm.at[pl.ds((base + ci * CHUNK) * D, CHUNK * D)], ssem)
    pending[(NCHUNK - 1) % 2].wait()
    pending[NCHUNK % 2].wait()


def kernel(input, weight):
    idx = input.reshape(-1).astype(jnp.int32)
    out = _sc_embed(idx, weight.reshape(-1))
    return out.reshape(input.shape + (weight.shape[1],))


# speedup vs baseline: 2.1637x; 2.1637x over previous
import functools

import jax
import jax.numpy as jnp
from jax import lax
from jax.experimental import pallas as pl
from jax.experimental.pallas import tpu as pltpu
from jax.experimental.pallas import tpu_sc as plsc

N = 4 * 8192
D = 512
NC, NS = 2, 16
NW = NC * NS
ROWS_PER_W = N // NW
CHUNK = 64
NCHUNK = ROWS_PER_W // CHUNK
GROUPS = CHUNK // 16
UNROLL = 8

_mesh = plsc.VectorSubcoreMesh(core_axis_name="c", subcore_axis_name="s")


@functools.partial(
    pl.kernel,
    mesh=_mesh,
    out_type=jax.ShapeDtypeStruct((N * D,), jnp.float32),
    scratch_types=[
        pltpu.VMEM((ROWS_PER_W,), jnp.int32),
        pltpu.VMEM((3 * D,), jnp.float32),
        pltpu.VMEM((CHUNK * D,), jnp.float32),
        pltpu.VMEM((CHUNK * D,), jnp.float32),
        pltpu.SemaphoreType.DMA,
    ],
    compiler_params=pltpu.CompilerParams(needs_layout_passes=False),
)
def _sc_embed(idx_hbm, table_hbm, out_hbm, idx_v, tab_v, rows_a, rows_b, ssem):
    wid = lax.axis_index("s") * NC + lax.axis_index("c")
    base = wid * ROWS_PER_W
    pltpu.sync_copy(idx_hbm.at[pl.ds(base, ROWS_PER_W)], idx_v)
    pltpu.sync_copy(table_hbm, tab_v)

    lane = jnp.arange(16, dtype=jnp.int32)
    lane_d = lane * D

    bufs = (rows_a, rows_b)
    pending = [None, None]
    for ci in range(NCHUNK):
        buf = bufs[ci % 2]
        if pending[ci % 2] is not None:
            pending[ci % 2].wait()
        for g in range(GROUPS):
            idxvec = idx_v[pl.ds(ci * CHUNK + g * 16, 16)]
            tbase = idxvec * D
            obase = lane_d + g * (16 * D)

            @plsc.parallel_loop(0, D, carry=(tbase, obase), unroll=UNROLL)
            def col_body(c, vecs, buf=buf):
                tvec, ovec = vecs
                vals = plsc.load_gather(tab_v, [tvec])
                plsc.store_scatter(buf, [ovec], vals)
                return tvec + 1, ovec + 1
        pending[ci % 2] = pltpu.async_copy(
            buf, out_hbm.at[pl.ds((base + ci * CHUNK) * D, CHUNK * D)], ssem)
    pending[(NCHUNK - 1) % 2].wait()
    pending[NCHUNK % 2].wait()


def kernel(input, weight):
    idx = input.reshape(-1).astype(jnp.int32)
    out = _sc_embed(idx, weight.reshape(-1))
    return out.reshape(input.shape + (weight.shape[1],))
